# Initial kernel scaffold; baseline (speedup 1.0000x reference)
import jax
import jax.numpy as jnp
from jax.experimental import pallas as pl


def kernel(x, word_embeddings, pos_embeddings, gamma, beta):
    raise NotImplementedError("write your pallas kernel here")


# speedup vs baseline: 2.0042x; 2.0042x over previous
import functools

import jax
import jax.numpy as jnp
from jax import lax
from jax.experimental import pallas as pl
from jax.experimental.pallas import tpu as pltpu
from jax.experimental.pallas import tpu_sc as plsc

VOCAB = 100000
MAXLEN = 200
EMBED = 128
BATCH = 1024
EPS = 1e-05

NC = 2
NS = 16
NW = NC * NS
ROWS_PER_W = BATCH // NW
NV = EMBED // 16


def _rsqrt(v):
    i = lax.bitcast_convert_type(v, jnp.int32)
    i = jnp.int32(0x5F3759DF) - (i >> 1)
    y = lax.bitcast_convert_type(i, jnp.float32)
    for _ in range(3):
        y = y * (1.5 - 0.5 * v * y * y)
    return y


def _ln_rows(rows_v, pos_v, g_v, b_v):
    gs = [g_v[pl.ds(k * 16, 16)] for k in range(NV)]
    bs = [b_v[pl.ds(k * 16, 16)] for k in range(NV)]

    def body(i, carry):
        xs = [rows_v[i, pl.ds(k * 16, 16)] + pos_v[i, pl.ds(k * 16, 16)]
              for k in range(NV)]
        s = xs[0]
        sq = xs[0] * xs[0]
        for k in range(1, NV):
            s = s + xs[k]
            sq = sq + xs[k] * xs[k]
        ssum = plsc.cumsum(s)[15]
        sqsum = plsc.cumsum(sq)[15]
        mean = ssum * (1.0 / EMBED)
        var = sqsum * (1.0 / EMBED) - mean * mean
        rstd = _rsqrt(var + EPS)
        for k in range(NV):
            a = gs[k] * rstd
            b = bs[k] - mean * a
            rows_v[i, pl.ds(k * 16, 16)] = xs[k] * a + b
        return carry

    lax.fori_loop(0, MAXLEN, body, 0)


def _body(x_hbm, wtab_hbm, pos_hbm, g_hbm, b_hbm, out_hbm,
          idx_v, rows_v, pos_v, g_v, b_v, gsem):
    wid = lax.axis_index("s") * NC + lax.axis_index("c")
    r0 = wid * ROWS_PER_W
    pltpu.sync_copy(pos_hbm, pos_v)
    pltpu.sync_copy(g_hbm, g_v)
    pltpu.sync_copy(b_hbm, b_v)

    def chunk(c, carry):
        r = r0 + c
        pltpu.sync_copy(x_hbm.at[r], idx_v)
        cp1 = pltpu.async_copy(wtab_hbm.at[idx_v.at[pl.ds(0, 128)]],
                               rows_v.at[pl.ds(0, 128)], gsem)
        cp2 = pltpu.async_copy(wtab_hbm.at[idx_v.at[pl.ds(128, 72)]],
                               rows_v.at[pl.ds(128, 72)], gsem)
        cp1.wait()
        cp2.wait()
        _ln_rows(rows_v, pos_v, g_v, b_v)
        pltpu.sync_copy(rows_v, out_hbm.at[r])
        return carry

    lax.fori_loop(0, ROWS_PER_W, chunk, 0)


def kernel(x, word_embeddings, pos_embeddings, gamma, beta):
    mesh = plsc.VectorSubcoreMesh(core_axis_name="c", subcore_axis_name="s",
                                  num_cores=NC, num_subcores=NS)
    f = pl.kernel(
        _body,
        out_type=jax.ShapeDtypeStruct((BATCH, MAXLEN, EMBED), jnp.float32),
        mesh=mesh,
        compiler_params=pltpu.CompilerParams(needs_layout_passes=False),
        scratch_types=[
            pltpu.VMEM((MAXLEN,), jnp.int32),
            pltpu.VMEM((MAXLEN, EMBED), jnp.float32),
            pltpu.VMEM((MAXLEN, EMBED), jnp.float32),
            pltpu.VMEM((EMBED,), jnp.float32),
            pltpu.VMEM((EMBED,), jnp.float32),
            pltpu.SemaphoreType.DMA,
        ],
    )
    return f(x, word_embeddings, pos_embeddings, gamma, beta)


# speedup vs baseline: 2.3663x; 1.1807x over previous
import functools

import jax
import jax.numpy as jnp
from jax import lax
from jax.experimental import pallas as pl
from jax.experimental.pallas import tpu as pltpu
from jax.experimental.pallas import tpu_sc as plsc

VOCAB = 100000
MAXLEN = 200
EMBED = 128
BATCH = 1024
EPS = 1e-05

NC = 2
NS = 16
NW = NC * NS
ROWS_PER_W = BATCH // NW
HALF = ROWS_PER_W // 2
NV = EMBED // 16


def _rsqrt(v):
    i = lax.bitcast_convert_type(v, jnp.int32)
    i = jnp.int32(0x5F3759DF) - (i >> 1)
    y = lax.bitcast_convert_type(i, jnp.float32)
    for _ in range(3):
        y = y * (1.5 - 0.5 * v * y * y)
    return y


def _ln_rows(rows_v, pos_v, gs, bs):
    def body(i, carry):
        xs = [rows_v[i, pl.ds(k * 16, 16)] + pos_v[i, pl.ds(k * 16, 16)]
              for k in range(NV)]
        s = xs[0]
        sq = xs[0] * xs[0]
        for k in range(1, NV):
            s = s + xs[k]
            sq = sq + xs[k] * xs[k]
        ssum = plsc.cumsum(s)[15]
        sqsum = plsc.cumsum(sq)[15]
        mean = ssum * (1.0 / EMBED)
        var = sqsum * (1.0 / EMBED) - mean * mean
        rstd = _rsqrt(var + EPS)
        for k in range(NV):
            a = gs[k] * rstd
            b = bs[k] - mean * a
            rows_v[i, pl.ds(k * 16, 16)] = xs[k] * a + b
        return carry

    lax.fori_loop(0, MAXLEN, body, 0)


def _start_gather(wtab_hbm, idx_v, rows_v, sem):
    pltpu.async_copy(wtab_hbm.at[idx_v.at[pl.ds(0, 128)]],
                     rows_v.at[pl.ds(0, 128)], sem)
    pltpu.async_copy(wtab_hbm.at[idx_v.at[pl.ds(128, 72)]],
                     rows_v.at[pl.ds(128, 72)], sem)


def _wait_gather(wtab_hbm, idx_v, rows_v, sem):
    pltpu.make_async_copy(wtab_hbm.at[idx_v.at[pl.ds(0, 128)]],
                          rows_v.at[pl.ds(0, 128)], sem).wait()
    pltpu.make_async_copy(wtab_hbm.at[idx_v.at[pl.ds(128, 72)]],
                          rows_v.at[pl.ds(128, 72)], sem).wait()


def _body(x_hbm, wtab_hbm, pos_hbm, g_hbm, b_hbm, out_hbm,
          idx_a, idx_b, rows_a, rows_b, pos_v, g_v, b_v,
          isa, isb, gsa, gsb, ssa, ssb):
    wid = lax.axis_index("s") * NC + lax.axis_index("c")
    r0 = wid * ROWS_PER_W
    pltpu.sync_copy(pos_hbm, pos_v)
    pltpu.sync_copy(g_hbm, g_v)
    pltpu.sync_copy(b_hbm, b_v)
    gs = [g_v[pl.ds(k * 16, 16)] for k in range(NV)]
    bs = [b_v[pl.ds(k * 16, 16)] for k in range(NV)]

    def wait_idx(idx_v, sem):
        pltpu.make_async_copy(x_hbm.at[r0], idx_v, sem).wait()

    def wait_store(rows_v, r, sem):
        pltpu.make_async_copy(rows_v, out_hbm.at[r], sem).wait()

    pltpu.async_copy(x_hbm.at[r0], idx_a, isa)
    wait_idx(idx_a, isa)
    _start_gather(wtab_hbm, idx_a, rows_a, gsa)
    pltpu.async_copy(x_hbm.at[r0 + 1], idx_b, isb)

    def body(i, carry):
        ca = r0 + 2 * i
        cb = ca + 1
        wait_idx(idx_b, isb)

        @pl.when(i > 0)
        def _():
            wait_store(rows_b, cb - 2, ssb)

        _start_gather(wtab_hbm, idx_b, rows_b, gsb)

        @pl.when(i < HALF - 1)
        def _():
            pltpu.async_copy(x_hbm.at[ca + 2], idx_a, isa)

        _wait_gather(wtab_hbm, idx_a, rows_a, gsa)
        _ln_rows(rows_a, pos_v, gs, bs)
        pltpu.async_copy(rows_a, out_hbm.at[ca], ssa)

        @pl.when(i < HALF - 1)
        def _():
            wait_idx(idx_a, isa)
            wait_store(rows_a, ca, ssa)
            _start_gather(wtab_hbm, idx_a, rows_a, gsa)
            pltpu.async_copy(x_hbm.at[cb + 2], idx_b, isb)

        _wait_gather(wtab_hbm, idx_b, rows_b, gsb)
        _ln_rows(rows_b, pos_v, gs, bs)
        pltpu.async_copy(rows_b, out_hbm.at[cb], ssb)
        return carry

    lax.fori_loop(0, HALF, body, 0)
    wait_store(rows_a, r0 + ROWS_PER_W - 2, ssa)
    wait_store(rows_b, r0 + ROWS_PER_W - 1, ssb)


def kernel(x, word_embeddings, pos_embeddings, gamma, beta):
    mesh = plsc.VectorSubcoreMesh(core_axis_name="c", subcore_axis_name="s",
                                  num_cores=NC, num_subcores=NS)
    f = pl.kernel(
        _body,
        out_type=jax.ShapeDtypeStruct((BATCH, MAXLEN, EMBED), jnp.float32),
        mesh=mesh,
        compiler_params=pltpu.CompilerParams(needs_layout_passes=False),
        scratch_types=[
            pltpu.VMEM((MAXLEN,), jnp.int32),
            pltpu.VMEM((MAXLEN,), jnp.int32),
            pltpu.VMEM((MAXLEN, EMBED), jnp.float32),
            pltpu.VMEM((MAXLEN, EMBED), jnp.float32),
            pltpu.VMEM((MAXLEN, EMBED), jnp.float32),
            pltpu.VMEM((EMBED,), jnp.float32),
            pltpu.VMEM((EMBED,), jnp.float32),
            pltpu.SemaphoreType.DMA,
            pltpu.SemaphoreType.DMA,
            pltpu.SemaphoreType.DMA,
            pltpu.SemaphoreType.DMA,
            pltpu.SemaphoreType.DMA,
            pltpu.SemaphoreType.DMA,
        ],
    )
    return f(x, word_embeddings, pos_embeddings, gamma, beta)


# speedup vs baseline: 4.5624x; 1.9280x over previous
import functools

import jax
import jax.numpy as jnp
from jax import lax
from jax.experimental import pallas as pl
from jax.experimental.pallas import tpu as pltpu
from jax.experimental.pallas import tpu_sc as plsc

VOCAB = 100000
MAXLEN = 200
EMBED = 128
BATCH = 1024
EPS = 1e-05

NC = 2
NS = 16
NW = NC * NS
ROWS_PER_W = BATCH // NW
HALF = ROWS_PER_W // 2
NV = EMBED // 16


def _rsqrt(v):
    i = lax.bitcast_convert_type(v, jnp.int32)
    i = jnp.int32(0x5F3759DF) - (i >> 1)
    y = lax.bitcast_convert_type(i, jnp.float32)
    for _ in range(2):
        y = y * (1.5 - 0.5 * v * y * y)
    return y


UNROLL = 4


def _ln_rows(rows_v, pos_v, gs, bs):
    def one_token(i):
        xs = [rows_v[i, pl.ds(k * 16, 16)] + pos_v[i, pl.ds(k * 16, 16)]
              for k in range(NV)]
        s = xs[0]
        sq = xs[0] * xs[0]
        for k in range(1, NV):
            s = s + xs[k]
            sq = sq + xs[k] * xs[k]
        ssum = plsc.cumsum(s)[15]
        sqsum = plsc.cumsum(sq)[15]
        mean = ssum * (1.0 / EMBED)
        var = sqsum * (1.0 / EMBED) - mean * mean
        rstd = _rsqrt(var + EPS)
        for k in range(NV):
            a = gs[k] * rstd
            b = bs[k] - mean * a
            rows_v[i, pl.ds(k * 16, 16)] = xs[k] * a + b

    def body(ii, carry):
        for u in range(UNROLL):
            one_token(ii * UNROLL + u)
        return carry

    lax.fori_loop(0, MAXLEN // UNROLL, body, 0)


def _start_gather(wtab_hbm, idx_v, rows_v, sem):
    pltpu.async_copy(wtab_hbm.at[idx_v.at[pl.ds(0, 128)]],
                     rows_v.at[pl.ds(0, 128)], sem)
    pltpu.async_copy(wtab_hbm.at[idx_v.at[pl.ds(128, 72)]],
                     rows_v.at[pl.ds(128, 72)], sem)


def _wait_gather(wtab_hbm, idx_v, rows_v, sem):
    pltpu.make_async_copy(wtab_hbm.at[idx_v.at[pl.ds(0, 128)]],
                          rows_v.at[pl.ds(0, 128)], sem).wait()
    pltpu.make_async_copy(wtab_hbm.at[idx_v.at[pl.ds(128, 72)]],
                          rows_v.at[pl.ds(128, 72)], sem).wait()


def _body(x_hbm, wtab_hbm, pos_hbm, g_hbm, b_hbm, out_hbm,
          idx_a, idx_b, rows_a, rows_b, pos_v, g_v, b_v,
          isa, isb, gsa, gsb, ssa, ssb):
    wid = lax.axis_index("s") * NC + lax.axis_index("c")
    r0 = wid * ROWS_PER_W
    pltpu.sync_copy(pos_hbm, pos_v)
    pltpu.sync_copy(g_hbm, g_v)
    pltpu.sync_copy(b_hbm, b_v)
    gs = [g_v[pl.ds(k * 16, 16)] for k in range(NV)]
    bs = [b_v[pl.ds(k * 16, 16)] for k in range(NV)]

    def wait_idx(idx_v, sem):
        pltpu.make_async_copy(x_hbm.at[r0], idx_v, sem).wait()

    def wait_store(rows_v, r, sem):
        pltpu.make_async_copy(rows_v, out_hbm.at[r], sem).wait()

    pltpu.async_copy(x_hbm.at[r0], idx_a, isa)
    wait_idx(idx_a, isa)
    _start_gather(wtab_hbm, idx_a, rows_a, gsa)
    pltpu.async_copy(x_hbm.at[r0 + 1], idx_b, isb)

    def body(i, carry):
        ca = r0 + 2 * i
        cb = ca + 1
        wait_idx(idx_b, isb)

        @pl.when(i > 0)
        def _():
            wait_store(rows_b, cb - 2, ssb)

        _start_gather(wtab_hbm, idx_b, rows_b, gsb)

        @pl.when(i < HALF - 1)
        def _():
            pltpu.async_copy(x_hbm.at[ca + 2], idx_a, isa)

        _wait_gather(wtab_hbm, idx_a, rows_a, gsa)
        _ln_rows(rows_a, pos_v, gs, bs)
        pltpu.async_copy(rows_a, out_hbm.at[ca], ssa)

        @pl.when(i < HALF - 1)
        def _():
            wait_idx(idx_a, isa)
            wait_store(rows_a, ca, ssa)
            _start_gather(wtab_hbm, idx_a, rows_a, gsa)
            pltpu.async_copy(x_hbm.at[cb + 2], idx_b, isb)

        _wait_gather(wtab_hbm, idx_b, rows_b, gsb)
        _ln_rows(rows_b, pos_v, gs, bs)
        pltpu.async_copy(rows_b, out_hbm.at[cb], ssb)
        return carry

    lax.fori_loop(0, HALF, body, 0)
    wait_store(rows_a, r0 + ROWS_PER_W - 2, ssa)
    wait_store(rows_b, r0 + ROWS_PER_W - 1, ssb)


def kernel(x, word_embeddings, pos_embeddings, gamma, beta):
    mesh = plsc.VectorSubcoreMesh(core_axis_name="c", subcore_axis_name="s",
                                  num_cores=NC, num_subcores=NS)
    f = pl.kernel(
        _body,
        out_type=jax.ShapeDtypeStruct((BATCH, MAXLEN, EMBED), jnp.float32),
        mesh=mesh,
        compiler_params=pltpu.CompilerParams(needs_layout_passes=False),
        scratch_types=[
            pltpu.VMEM((MAXLEN,), jnp.int32),
            pltpu.VMEM((MAXLEN,), jnp.int32),
            pltpu.VMEM((MAXLEN, EMBED), jnp.float32),
            pltpu.VMEM((MAXLEN, EMBED), jnp.float32),
            pltpu.VMEM((MAXLEN, EMBED), jnp.float32),
            pltpu.VMEM((EMBED,), jnp.float32),
            pltpu.VMEM((EMBED,), jnp.float32),
            pltpu.SemaphoreType.DMA,
            pltpu.SemaphoreType.DMA,
            pltpu.SemaphoreType.DMA,
            pltpu.SemaphoreType.DMA,
            pltpu.SemaphoreType.DMA,
            pltpu.SemaphoreType.DMA,
        ],
    )
    return f(x, word_embeddings, pos_embeddings, gamma, beta)


# speedup vs baseline: 5.3976x; 1.1831x over previous
import functools

import jax
import jax.numpy as jnp
from jax import lax
from jax.experimental import pallas as pl
from jax.experimental.pallas import tpu as pltpu
from jax.experimental.pallas import tpu_sc as plsc

VOCAB = 100000
MAXLEN = 200
EMBED = 128
BATCH = 1024
EPS = 1e-05

NC = 2
NS = 16
NW = NC * NS
ROWS_PER_W = BATCH // NW
HALF = ROWS_PER_W // 2
NV = EMBED // 16


def _rsqrt(v):
    i = lax.bitcast_convert_type(v, jnp.int32)
    i = jnp.int32(0x5F3759DF) - (i >> 1)
    y = lax.bitcast_convert_type(i, jnp.float32)
    for _ in range(2):
        y = y * (1.5 - 0.5 * v * y * y)
    return y


UNROLL = 8


def _ln_rows(rows_v, pos_v, gs, bs):
    def one_token(i):
        xs = [rows_v[i, pl.ds(k * 16, 16)] + pos_v[i, pl.ds(k * 16, 16)]
              for k in range(NV)]
        s = xs[0]
        sq = xs[0] * xs[0]
        for k in range(1, NV):
            s = s + xs[k]
            sq = sq + xs[k] * xs[k]
        ssum = plsc.cumsum(s)[15]
        sqsum = plsc.cumsum(sq)[15]
        mean = ssum * (1.0 / EMBED)
        var = sqsum * (1.0 / EMBED) - mean * mean
        rstd = _rsqrt(var + EPS)
        for k in range(NV):
            a = gs[k] * rstd
            b = bs[k] - mean * a
            rows_v[i, pl.ds(k * 16, 16)] = xs[k] * a + b

    def body(ii, carry):
        for u in range(UNROLL):
            one_token(ii * UNROLL + u)
        return carry

    lax.fori_loop(0, MAXLEN // UNROLL, body, 0)


def _start_gather(wtab_hbm, idx_v, rows_v, sem):
    pltpu.async_copy(wtab_hbm.at[idx_v.at[pl.ds(0, 128)]],
                     rows_v.at[pl.ds(0, 128)], sem)
    pltpu.async_copy(wtab_hbm.at[idx_v.at[pl.ds(128, 72)]],
                     rows_v.at[pl.ds(128, 72)], sem)


def _wait_gather(wtab_hbm, idx_v, rows_v, sem):
    pltpu.make_async_copy(wtab_hbm.at[idx_v.at[pl.ds(0, 128)]],
                          rows_v.at[pl.ds(0, 128)], sem).wait()
    pltpu.make_async_copy(wtab_hbm.at[idx_v.at[pl.ds(128, 72)]],
                          rows_v.at[pl.ds(128, 72)], sem).wait()


def _body(x_hbm, wtab_hbm, pos_hbm, g_hbm, b_hbm, out_hbm,
          idx_a, idx_b, rows_a, rows_b, pos_v, g_v, b_v,
          isa, isb, gsa, gsb, ssa, ssb):
    wid = lax.axis_index("s") * NC + lax.axis_index("c")
    r0 = wid * ROWS_PER_W
    pltpu.sync_copy(pos_hbm, pos_v)
    pltpu.sync_copy(g_hbm, g_v)
    pltpu.sync_copy(b_hbm, b_v)
    gs = [g_v[pl.ds(k * 16, 16)] for k in range(NV)]
    bs = [b_v[pl.ds(k * 16, 16)] for k in range(NV)]

    def wait_idx(idx_v, sem):
        pltpu.make_async_copy(x_hbm.at[r0], idx_v, sem).wait()

    def wait_store(rows_v, r, sem):
        pltpu.make_async_copy(rows_v, out_hbm.at[r], sem).wait()

    pltpu.async_copy(x_hbm.at[r0], idx_a, isa)
    wait_idx(idx_a, isa)
    _start_gather(wtab_hbm, idx_a, rows_a, gsa)
    pltpu.async_copy(x_hbm.at[r0 + 1], idx_b, isb)

    def body(i, carry):
        ca = r0 + 2 * i
        cb = ca + 1
        wait_idx(idx_b, isb)

        @pl.when(i > 0)
        def _():
            wait_store(rows_b, cb - 2, ssb)

        _start_gather(wtab_hbm, idx_b, rows_b, gsb)

        @pl.when(i < HALF - 1)
        def _():
            pltpu.async_copy(x_hbm.at[ca + 2], idx_a, isa)

        _wait_gather(wtab_hbm, idx_a, rows_a, gsa)
        _ln_rows(rows_a, pos_v, gs, bs)
        pltpu.async_copy(rows_a, out_hbm.at[ca], ssa)

        @pl.when(i < HALF - 1)
        def _():
            wait_idx(idx_a, isa)
            wait_store(rows_a, ca, ssa)
            _start_gather(wtab_hbm, idx_a, rows_a, gsa)
            pltpu.async_copy(x_hbm.at[cb + 2], idx_b, isb)

        _wait_gather(wtab_hbm, idx_b, rows_b, gsb)
        _ln_rows(rows_b, pos_v, gs, bs)
        pltpu.async_copy(rows_b, out_hbm.at[cb], ssb)
        return carry

    lax.fori_loop(0, HALF, body, 0)
    wait_store(rows_a, r0 + ROWS_PER_W - 2, ssa)
    wait_store(rows_b, r0 + ROWS_PER_W - 1, ssb)


def kernel(x, word_embeddings, pos_embeddings, gamma, beta):
    mesh = plsc.VectorSubcoreMesh(core_axis_name="c", subcore_axis_name="s",
                                  num_cores=NC, num_subcores=NS)
    f = pl.kernel(
        _body,
        out_type=jax.ShapeDtypeStruct((BATCH, MAXLEN, EMBED), jnp.float32),
        mesh=mesh,
        compiler_params=pltpu.CompilerParams(needs_layout_passes=False),
        scratch_types=[
            pltpu.VMEM((MAXLEN,), jnp.int32),
            pltpu.VMEM((MAXLEN,), jnp.int32),
            pltpu.VMEM((MAXLEN, EMBED), jnp.float32),
            pltpu.VMEM((MAXLEN, EMBED), jnp.float32),
            pltpu.VMEM((MAXLEN, EMBED), jnp.float32),
            pltpu.VMEM((EMBED,), jnp.float32),
            pltpu.VMEM((EMBED,), jnp.float32),
            pltpu.SemaphoreType.DMA,
            pltpu.SemaphoreType.DMA,
            pltpu.SemaphoreType.DMA,
            pltpu.SemaphoreType.DMA,
            pltpu.SemaphoreType.DMA,
            pltpu.SemaphoreType.DMA,
        ],
    )
    return f(x, word_embeddings, pos_embeddings, gamma, beta)
